# baseline (speedup 1.0000x reference)
import functools

import jax
import jax.numpy as jnp
import numpy as np
from jax.experimental import pallas as pl
from jax.experimental.pallas import tpu as pltpu

_NC = 6
_TOPK = 10
_ALPHA = 0.5
_BETA = 6.0
_BOX_W = 7.5
_CLS_W = 0.5
_INNER_W = 0.5
_SCALE = 0.7
_LEVELS = [(80, 80), (40, 40), (20, 20)]
_A = sum(h * w for h, w in _LEVELS)
_M = 40
_EPS = 1e-9


def _anchors_np():
    pts = []
    for (H, W) in _LEVELS:
        gy, gx = np.meshgrid(np.arange(H), np.arange(W), indexing="ij")
        ax = (gx + 0.5) / W
        ay = (gy + 0.5) / H
        pts.append(np.stack([ax.reshape(-1), ay.reshape(-1)], 0))
    return np.concatenate(pts, 1).astype(np.float32)


def _loss_kernel(cls_ref, box_ref, tgt_ref, anc_ref, out_ref):
    b = pl.program_id(0)
    nb = pl.num_programs(0)

    @pl.when(b == 0)
    def _init():
        out_ref[...] = jnp.zeros_like(out_ref)

    x = cls_ref[0]
    ps = jax.nn.sigmoid(x)
    bx = jnp.clip(jax.nn.sigmoid(box_ref[0]), 1e-4, 1.0 - 1e-4)
    pc = bx[0:2, :]
    ph2 = bx[2:4, :] / 2
    plo = pc - ph2
    phi = pc + ph2
    px1, py1 = plo[0:1, :], plo[1:2, :]
    px2, py2 = phi[0:1, :], phi[1:2, :]

    tgt = tgt_ref[0]
    gl = jnp.clip(jnp.floor(tgt[:, 0:1]), 0.0, _NC - 1.0)
    gcx, gcy = tgt[:, 1:2], tgt[:, 2:3]
    gw, gh = tgt[:, 3:4], tgt[:, 4:5]
    gx1, gy1 = gcx - gw / 2, gcy - gh / 2
    gx2, gy2 = gcx + gw / 2, gcy + gh / 2

    ax = anc_ref[0, 0:1, :]
    ay = anc_ref[0, 1:2, :]

    d1 = jnp.minimum(ax - gx1, ay - gy1)
    d2 = jnp.minimum(gx2 - ax, gy2 - ay)
    mask_in = jnp.minimum(d1, d2) > _EPS

    ltx = jnp.maximum(gx1, px1)
    lty = jnp.maximum(gy1, py1)
    rbx = jnp.minimum(gx2, px2)
    rby = jnp.minimum(gy2, py2)
    iw = jnp.clip(rbx - ltx, 0.0, None)
    ih = jnp.clip(rby - lty, 0.0, None)
    inter = iw * ih
    a1 = jnp.clip(gx2 - gx1, 0.0, None) * jnp.clip(gy2 - gy1, 0.0, None)
    a2 = jnp.clip(px2 - px1, 0.0, None) * jnp.clip(py2 - py1, 0.0, None)
    overlaps = inter / (a1 + a2 - inter + 1e-7)

    sq = jnp.sqrt(ps)
    sqscore = jnp.where(
        gl == 0.0, sq[0:1, :],
        jnp.where(gl == 1.0, sq[1:2, :],
                  jnp.where(gl == 2.0, sq[2:3, :],
                            jnp.where(gl == 3.0, sq[3:4, :],
                                      jnp.where(gl == 4.0, sq[4:5, :],
                                                sq[5:6, :])))))

    o2 = overlaps * overlaps
    align = sqscore * (o2 * o2 * o2)
    metric = jnp.where(mask_in, align, 0.0)

    v = metric
    for _ in range(_TOPK - 1):
        cur = jnp.max(v, axis=1, keepdims=True)
        v = jnp.where(v == cur, -1.0, v)
    cur_last = jnp.max(v, axis=1, keepdims=True)
    mask_pos0 = (((v < 0.0) | (v == cur_last))
                 & (metric > _EPS)).astype(jnp.float32)

    fg_cnt = jnp.sum(mask_pos0, axis=0, keepdims=True)
    multi = fg_cnt > 1.0

    iota_m = jax.lax.broadcasted_iota(jnp.int32, (_M, _A), 0)
    mo = jnp.max(overlaps, axis=0, keepdims=True)
    eqo = overlaps == mo
    fm = jnp.min(jnp.where(eqo, iota_m, _M), axis=0, keepdims=True)
    is_max = (iota_m == fm).astype(jnp.float32)

    mask_pos = jnp.where(multi, is_max, mask_pos0)

    fp = jnp.min(jnp.where(mask_pos > 0.0, iota_m, _M), axis=0, keepdims=True)
    fg = fp < _M
    fgf = fg.astype(jnp.float32)
    tgt_idx = jnp.where(fg, fp, 0)
    sel = (iota_m == tgt_idx).astype(jnp.float32)

    t_lab = jnp.sum(sel * gl, axis=0, keepdims=True)
    tcx = jnp.sum(sel * gcx, axis=0, keepdims=True)
    tcy = jnp.sum(sel * gcy, axis=0, keepdims=True)
    tw = jnp.sum(sel * gw, axis=0, keepdims=True)
    th = jnp.sum(sel * gh, axis=0, keepdims=True)

    align_pos = align * mask_pos
    pos_align = jnp.max(align_pos, axis=1, keepdims=True)
    pos_ov = jnp.max(overlaps * mask_pos, axis=1, keepdims=True)
    ratio = pos_ov / (pos_align + _EPS)
    norm_max = jnp.max(align_pos * ratio, axis=0, keepdims=True)

    tmag = fgf * norm_max
    iota_c = jax.lax.broadcasted_iota(jnp.int32, (_NC, _A), 0).astype(jnp.float32)
    t_mat = jnp.where(iota_c == t_lab, tmag, 0.0)
    bce = jnp.clip(x, 0.0, None) - x * t_mat + jnp.log1p(jnp.exp(-jnp.abs(x)))
    bce_sum = jnp.sum(bce)

    def iou_2a(lo1, hi1, lo2, hi2):
        jwh = jnp.clip(jnp.minimum(hi1, hi2) - jnp.maximum(lo1, lo2), 0.0, None)
        ji = jwh[0:1, :] * jwh[1:2, :]
        wh1 = jnp.clip(hi1 - lo1, 0.0, None)
        wh2 = jnp.clip(hi2 - lo2, 0.0, None)
        aa = wh1[0:1, :] * wh1[1:2, :]
        ab = wh2[0:1, :] * wh2[1:2, :]
        return ji / (aa + ab - ji + 1e-7)

    tc = jnp.concatenate([tcx, tcy], 0)
    th2 = jnp.concatenate([tw, th], 0) / 2
    tlo, thi = tc - th2, tc + th2
    iou = iou_2a(plo, phi, tlo, thi)
    ph2s, th2s = ph2 * _SCALE, th2 * _SCALE
    inner = iou_2a(pc - ph2s, pc + ph2s, tc - th2s, tc + th2s)
    per = (1.0 - iou) + _INNER_W * (1.0 - inner)
    box_sum = jnp.sum(per * fgf)
    fg_sum = jnp.sum(fgf)

    li = jax.lax.broadcasted_iota(jnp.int32, (1, 128), 1)
    contrib = (jnp.where(li == 0, fg_sum, 0.0)
               + jnp.where(li == 1, bce_sum, 0.0)
               + jnp.where(li == 2, box_sum, 0.0))
    out_ref[...] += contrib

    @pl.when(b == nb - 1)
    def _fin():
        acc = out_ref[...]
        num_fg = jnp.clip(acc[0:1, 0:1], 1.0, None)
        cls_loss = acc[0:1, 1:2] / num_fg
        box_loss = acc[0:1, 2:3] / num_fg
        total = _CLS_W * cls_loss + _BOX_W * box_loss
        fin = (jnp.where(li == 3, total, 0.0)
               + jnp.where(li == 4, cls_loss, 0.0)
               + jnp.where(li == 5, box_loss, 0.0))
        out_ref[...] = acc + fin


@jax.jit
def kernel(p3, p4, p5, targets):
    B = p3.shape[0]
    cls_in = jnp.concatenate(
        [p[:, :_NC].reshape(B, _NC, -1) for p in (p3, p4, p5)], axis=2)
    box_in = jnp.concatenate(
        [p[:, _NC + 1:].reshape(B, 4, -1) for p in (p3, p4, p5)], axis=2)
    anc = jnp.asarray(_anchors_np())[None]

    out = pl.pallas_call(
        _loss_kernel,
        grid=(B,),
        in_specs=[
            pl.BlockSpec((1, _NC, _A), lambda b: (b, 0, 0)),
            pl.BlockSpec((1, 4, _A), lambda b: (b, 0, 0)),
            pl.BlockSpec((1, _M, 5), lambda b: (b, 0, 0)),
            pl.BlockSpec((1, 2, _A), lambda b: (0, 0, 0)),
        ],
        out_specs=pl.BlockSpec((1, 128), lambda b: (0, 0)),
        out_shape=jax.ShapeDtypeStruct((1, 128), jnp.float32),
    )(cls_in, box_in, targets, anc)

    total = out[0, 3]
    cls_loss = out[0, 4]
    box_loss = out[0, 5]
    return total, cls_loss, box_loss


# speedup vs baseline: 1.0196x; 1.0196x over previous
import functools

import jax
import jax.numpy as jnp
import numpy as np
from jax.experimental import pallas as pl
from jax.experimental.pallas import tpu as pltpu

_NC = 6
_TOPK = 10
_ALPHA = 0.5
_BETA = 6.0
_BOX_W = 7.5
_CLS_W = 0.5
_INNER_W = 0.5
_SCALE = 0.7
_LEVELS = [(80, 80), (40, 40), (20, 20)]
_A = sum(h * w for h, w in _LEVELS)
_M = 40
_EPS = 1e-9


def _anchors_np():
    pts = []
    for (H, W) in _LEVELS:
        gy, gx = np.meshgrid(np.arange(H), np.arange(W), indexing="ij")
        ax = (gx + 0.5) / W
        ay = (gy + 0.5) / H
        pts.append(np.stack([ax.reshape(-1), ay.reshape(-1)], 0))
    return np.concatenate(pts, 1).astype(np.float32)


def _loss_kernel(cls_ref, box_ref, tgt_ref, anc_ref, out_ref):
    b = pl.program_id(0)
    nb = pl.num_programs(0)

    @pl.when(b == 0)
    def _init():
        out_ref[...] = jnp.zeros_like(out_ref)

    x = cls_ref[0]
    ps = jax.nn.sigmoid(x)
    bx = jnp.clip(jax.nn.sigmoid(box_ref[0]), 1e-4, 1.0 - 1e-4)
    pc = bx[0:2, :]
    ph2 = bx[2:4, :] / 2
    plo = pc - ph2
    phi = pc + ph2
    px1, py1 = plo[0:1, :], plo[1:2, :]
    px2, py2 = phi[0:1, :], phi[1:2, :]

    tgt = tgt_ref[0]
    gl = jnp.clip(jnp.floor(tgt[:, 0:1]), 0.0, _NC - 1.0)
    gcx, gcy = tgt[:, 1:2], tgt[:, 2:3]
    gw, gh = tgt[:, 3:4], tgt[:, 4:5]
    gx1, gy1 = gcx - gw / 2, gcy - gh / 2
    gx2, gy2 = gcx + gw / 2, gcy + gh / 2

    ax = anc_ref[0, 0:1, :]
    ay = anc_ref[0, 1:2, :]

    d1 = jnp.minimum(ax - gx1, ay - gy1)
    d2 = jnp.minimum(gx2 - ax, gy2 - ay)
    mask_in = jnp.minimum(d1, d2) > _EPS

    ltx = jnp.maximum(gx1, px1)
    lty = jnp.maximum(gy1, py1)
    rbx = jnp.minimum(gx2, px2)
    rby = jnp.minimum(gy2, py2)
    iw = jnp.clip(rbx - ltx, 0.0, None)
    ih = jnp.clip(rby - lty, 0.0, None)
    inter = iw * ih
    a1 = jnp.clip(gx2 - gx1, 0.0, None) * jnp.clip(gy2 - gy1, 0.0, None)
    a2 = jnp.clip(px2 - px1, 0.0, None) * jnp.clip(py2 - py1, 0.0, None)
    overlaps = inter / (a1 + a2 - inter + 1e-7)

    sq = jnp.sqrt(ps)
    sqscore = jnp.where(
        gl == 0.0, sq[0:1, :],
        jnp.where(gl == 1.0, sq[1:2, :],
                  jnp.where(gl == 2.0, sq[2:3, :],
                            jnp.where(gl == 3.0, sq[3:4, :],
                                      jnp.where(gl == 4.0, sq[4:5, :],
                                                sq[5:6, :])))))

    o2 = overlaps * overlaps
    align = sqscore * (o2 * o2 * o2)
    metric = jnp.where(mask_in, align, 0.0)

    v = metric
    for _ in range(_TOPK):
        cur = jnp.max(v, axis=1, keepdims=True)
        v = jnp.where(v == cur, -1.0, v)
    mask_pos0 = ((v < 0.0) & (metric > _EPS)).astype(jnp.float32)

    fg_cnt = jnp.sum(mask_pos0, axis=0, keepdims=True)
    multi = fg_cnt > 1.0

    iota_m = jax.lax.broadcasted_iota(jnp.int32, (_M, _A), 0)
    mo = jnp.max(overlaps, axis=0, keepdims=True)
    eqo = overlaps == mo
    fm = jnp.min(jnp.where(eqo, iota_m, _M), axis=0, keepdims=True)
    is_max = (iota_m == fm).astype(jnp.float32)

    mask_pos = jnp.where(multi, is_max, mask_pos0)

    fp = jnp.min(jnp.where(mask_pos > 0.0, iota_m, _M), axis=0, keepdims=True)
    fg = fp < _M
    fgf = fg.astype(jnp.float32)
    tgt_idx = jnp.where(fg, fp, 0)
    sel = (iota_m == tgt_idx).astype(jnp.float32)

    t_lab = jnp.sum(sel * gl, axis=0, keepdims=True)
    tcx = jnp.sum(sel * gcx, axis=0, keepdims=True)
    tcy = jnp.sum(sel * gcy, axis=0, keepdims=True)
    tw = jnp.sum(sel * gw, axis=0, keepdims=True)
    th = jnp.sum(sel * gh, axis=0, keepdims=True)

    align_pos = align * mask_pos
    pos_align = jnp.max(align_pos, axis=1, keepdims=True)
    pos_ov = jnp.max(overlaps * mask_pos, axis=1, keepdims=True)
    ratio = pos_ov / (pos_align + _EPS)
    norm_max = jnp.max(align_pos * ratio, axis=0, keepdims=True)

    tmag = fgf * norm_max
    iota_c = jax.lax.broadcasted_iota(jnp.int32, (_NC, _A), 0).astype(jnp.float32)
    t_mat = jnp.where(iota_c == t_lab, tmag, 0.0)
    bce = jnp.clip(x, 0.0, None) - x * t_mat + jnp.log1p(jnp.exp(-jnp.abs(x)))
    bce_sum = jnp.sum(bce)

    def iou_2a(lo1, hi1, lo2, hi2):
        jwh = jnp.clip(jnp.minimum(hi1, hi2) - jnp.maximum(lo1, lo2), 0.0, None)
        ji = jwh[0:1, :] * jwh[1:2, :]
        wh1 = jnp.clip(hi1 - lo1, 0.0, None)
        wh2 = jnp.clip(hi2 - lo2, 0.0, None)
        aa = wh1[0:1, :] * wh1[1:2, :]
        ab = wh2[0:1, :] * wh2[1:2, :]
        return ji / (aa + ab - ji + 1e-7)

    tc = jnp.concatenate([tcx, tcy], 0)
    th2 = jnp.concatenate([tw, th], 0) / 2
    tlo, thi = tc - th2, tc + th2
    iou = iou_2a(plo, phi, tlo, thi)
    ph2s, th2s = ph2 * _SCALE, th2 * _SCALE
    inner = iou_2a(pc - ph2s, pc + ph2s, tc - th2s, tc + th2s)
    per = (1.0 - iou) + _INNER_W * (1.0 - inner)
    box_sum = jnp.sum(per * fgf)
    fg_sum = jnp.sum(fgf)

    li = jax.lax.broadcasted_iota(jnp.int32, (1, 128), 1)
    contrib = (jnp.where(li == 0, fg_sum, 0.0)
               + jnp.where(li == 1, bce_sum, 0.0)
               + jnp.where(li == 2, box_sum, 0.0))
    out_ref[...] += contrib

    @pl.when(b == nb - 1)
    def _fin():
        acc = out_ref[...]
        num_fg = jnp.clip(acc[0:1, 0:1], 1.0, None)
        cls_loss = acc[0:1, 1:2] / num_fg
        box_loss = acc[0:1, 2:3] / num_fg
        total = _CLS_W * cls_loss + _BOX_W * box_loss
        fin = (jnp.where(li == 3, total, 0.0)
               + jnp.where(li == 4, cls_loss, 0.0)
               + jnp.where(li == 5, box_loss, 0.0))
        out_ref[...] = acc + fin


@jax.jit
def kernel(p3, p4, p5, targets):
    B = p3.shape[0]
    cls_in = jnp.concatenate(
        [p[:, :_NC].reshape(B, _NC, -1) for p in (p3, p4, p5)], axis=2)
    box_in = jnp.concatenate(
        [p[:, _NC + 1:].reshape(B, 4, -1) for p in (p3, p4, p5)], axis=2)
    anc = jnp.asarray(_anchors_np())[None]

    out = pl.pallas_call(
        _loss_kernel,
        grid=(B,),
        in_specs=[
            pl.BlockSpec((1, _NC, _A), lambda b: (b, 0, 0)),
            pl.BlockSpec((1, 4, _A), lambda b: (b, 0, 0)),
            pl.BlockSpec((1, _M, 5), lambda b: (b, 0, 0)),
            pl.BlockSpec((1, 2, _A), lambda b: (0, 0, 0)),
        ],
        out_specs=pl.BlockSpec((1, 128), lambda b: (0, 0)),
        out_shape=jax.ShapeDtypeStruct((1, 128), jnp.float32),
    )(cls_in, box_in, targets, anc)

    total = out[0, 3]
    cls_loss = out[0, 4]
    box_loss = out[0, 5]
    return total, cls_loss, box_loss


# speedup vs baseline: 1.0729x; 1.0523x over previous
import functools

import jax
import jax.numpy as jnp
import numpy as np
from jax.experimental import pallas as pl
from jax.experimental.pallas import tpu as pltpu

_NC = 6
_TOPK = 10
_ALPHA = 0.5
_BETA = 6.0
_BOX_W = 7.5
_CLS_W = 0.5
_INNER_W = 0.5
_SCALE = 0.7
_LEVELS = [(80, 80), (40, 40), (20, 20)]
_A = sum(h * w for h, w in _LEVELS)
_M = 40
_EPS = 1e-9


def _anchors_np():
    pts = []
    for (H, W) in _LEVELS:
        gy, gx = np.meshgrid(np.arange(H), np.arange(W), indexing="ij")
        ax = (gx + 0.5) / W
        ay = (gy + 0.5) / H
        pts.append(np.stack([ax.reshape(-1), ay.reshape(-1)], 0))
    return np.concatenate(pts, 1).astype(np.float32)


def _loss_kernel(cls_ref, box_ref, tgt_ref, anc_ref, out_ref):
    b = pl.program_id(0)
    nb = pl.num_programs(0)

    @pl.when(b == 0)
    def _init():
        out_ref[...] = jnp.zeros_like(out_ref)

    x = cls_ref[0]
    ps = jax.nn.sigmoid(x)
    bx = jnp.clip(jax.nn.sigmoid(box_ref[0]), 1e-4, 1.0 - 1e-4)
    pc = bx[0:2, :]
    ph2 = bx[2:4, :] / 2
    plo = pc - ph2
    phi = pc + ph2
    px1, py1 = plo[0:1, :], plo[1:2, :]
    px2, py2 = phi[0:1, :], phi[1:2, :]

    tgt = tgt_ref[0]
    gl = jnp.clip(jnp.floor(tgt[:, 0:1]), 0.0, _NC - 1.0)
    gcx, gcy = tgt[:, 1:2], tgt[:, 2:3]
    gw, gh = tgt[:, 3:4], tgt[:, 4:5]
    gx1, gy1 = gcx - gw / 2, gcy - gh / 2
    gx2, gy2 = gcx + gw / 2, gcy + gh / 2

    ax = anc_ref[0, 0:1, :]
    ay = anc_ref[0, 1:2, :]

    d1 = jnp.minimum(ax - gx1, ay - gy1)
    d2 = jnp.minimum(gx2 - ax, gy2 - ay)
    mask_in = jnp.minimum(d1, d2) > _EPS

    ltx = jnp.maximum(gx1, px1)
    lty = jnp.maximum(gy1, py1)
    rbx = jnp.minimum(gx2, px2)
    rby = jnp.minimum(gy2, py2)
    iw = jnp.clip(rbx - ltx, 0.0, None)
    ih = jnp.clip(rby - lty, 0.0, None)
    inter = iw * ih
    a1 = jnp.clip(gx2 - gx1, 0.0, None) * jnp.clip(gy2 - gy1, 0.0, None)
    a2 = jnp.clip(px2 - px1, 0.0, None) * jnp.clip(py2 - py1, 0.0, None)
    overlaps = inter / (a1 + a2 - inter + 1e-7)

    sq = jnp.sqrt(ps)
    sqscore = jnp.where(
        gl == 0.0, sq[0:1, :],
        jnp.where(gl == 1.0, sq[1:2, :],
                  jnp.where(gl == 2.0, sq[2:3, :],
                            jnp.where(gl == 3.0, sq[3:4, :],
                                      jnp.where(gl == 4.0, sq[4:5, :],
                                                sq[5:6, :])))))

    o2 = overlaps * overlaps
    align = sqscore * (o2 * o2 * o2)
    metric = jnp.where(mask_in, align, 0.0)

    v = metric
    for _ in range(_TOPK):
        cur = jnp.max(v, axis=1, keepdims=True)
        v = jnp.where(v == cur, -1.0, v)
    mask_pos0 = ((v < 0.0) & (metric > _EPS)).astype(jnp.float32)

    fg_cnt = jnp.sum(mask_pos0, axis=0, keepdims=True)
    multi = fg_cnt > 1.0
    fg = fg_cnt > 0.0
    fgf = fg.astype(jnp.float32)

    mo = jnp.max(overlaps, axis=0, keepdims=True)
    is_max = (overlaps == mo).astype(jnp.float32)
    mask_pos = jnp.where(multi, is_max, mask_pos0)

    t_lab = jnp.sum(mask_pos * gl, axis=0, keepdims=True)
    tcx = jnp.sum(mask_pos * gcx, axis=0, keepdims=True)
    tcy = jnp.sum(mask_pos * gcy, axis=0, keepdims=True)
    tw = jnp.sum(mask_pos * gw, axis=0, keepdims=True)
    th = jnp.sum(mask_pos * gh, axis=0, keepdims=True)

    align_pos = align * mask_pos
    pos_align = jnp.max(align_pos, axis=1, keepdims=True)
    pos_ov = jnp.max(overlaps * mask_pos, axis=1, keepdims=True)
    ratio = pos_ov / (pos_align + _EPS)
    norm_max = jnp.max(align_pos * ratio, axis=0, keepdims=True)

    tmag = fgf * norm_max
    iota_c = jax.lax.broadcasted_iota(jnp.int32, (_NC, _A), 0).astype(jnp.float32)
    t_mat = jnp.where(iota_c == t_lab, tmag, 0.0)
    bce = jnp.clip(x, 0.0, None) - x * t_mat + jnp.log1p(jnp.exp(-jnp.abs(x)))
    bce_sum = jnp.sum(bce)

    def iou_2a(lo1, hi1, lo2, hi2):
        jwh = jnp.clip(jnp.minimum(hi1, hi2) - jnp.maximum(lo1, lo2), 0.0, None)
        ji = jwh[0:1, :] * jwh[1:2, :]
        wh1 = jnp.clip(hi1 - lo1, 0.0, None)
        wh2 = jnp.clip(hi2 - lo2, 0.0, None)
        aa = wh1[0:1, :] * wh1[1:2, :]
        ab = wh2[0:1, :] * wh2[1:2, :]
        return ji / (aa + ab - ji + 1e-7)

    tc = jnp.concatenate([tcx, tcy], 0)
    th2 = jnp.concatenate([tw, th], 0) / 2
    tlo, thi = tc - th2, tc + th2
    iou = iou_2a(plo, phi, tlo, thi)
    ph2s, th2s = ph2 * _SCALE, th2 * _SCALE
    inner = iou_2a(pc - ph2s, pc + ph2s, tc - th2s, tc + th2s)
    per = (1.0 - iou) + _INNER_W * (1.0 - inner)
    box_sum = jnp.sum(per * fgf)
    fg_sum = jnp.sum(fgf)

    li = jax.lax.broadcasted_iota(jnp.int32, (1, 128), 1)
    contrib = (jnp.where(li == 0, fg_sum, 0.0)
               + jnp.where(li == 1, bce_sum, 0.0)
               + jnp.where(li == 2, box_sum, 0.0))
    out_ref[...] += contrib

    @pl.when(b == nb - 1)
    def _fin():
        acc = out_ref[...]
        num_fg = jnp.clip(acc[0:1, 0:1], 1.0, None)
        cls_loss = acc[0:1, 1:2] / num_fg
        box_loss = acc[0:1, 2:3] / num_fg
        total = _CLS_W * cls_loss + _BOX_W * box_loss
        fin = (jnp.where(li == 3, total, 0.0)
               + jnp.where(li == 4, cls_loss, 0.0)
               + jnp.where(li == 5, box_loss, 0.0))
        out_ref[...] = acc + fin


@jax.jit
def kernel(p3, p4, p5, targets):
    B = p3.shape[0]
    cls_in = jnp.concatenate(
        [p[:, :_NC].reshape(B, _NC, -1) for p in (p3, p4, p5)], axis=2)
    box_in = jnp.concatenate(
        [p[:, _NC + 1:].reshape(B, 4, -1) for p in (p3, p4, p5)], axis=2)
    anc = jnp.asarray(_anchors_np())[None]

    out = pl.pallas_call(
        _loss_kernel,
        grid=(B,),
        in_specs=[
            pl.BlockSpec((1, _NC, _A), lambda b: (b, 0, 0)),
            pl.BlockSpec((1, 4, _A), lambda b: (b, 0, 0)),
            pl.BlockSpec((1, _M, 5), lambda b: (b, 0, 0)),
            pl.BlockSpec((1, 2, _A), lambda b: (0, 0, 0)),
        ],
        out_specs=pl.BlockSpec((1, 128), lambda b: (0, 0)),
        out_shape=jax.ShapeDtypeStruct((1, 128), jnp.float32),
    )(cls_in, box_in, targets, anc)

    total = out[0, 3]
    cls_loss = out[0, 4]
    box_loss = out[0, 5]
    return total, cls_loss, box_loss


# speedup vs baseline: 1.0926x; 1.0183x over previous
import functools

import jax
import jax.numpy as jnp
import numpy as np
from jax.experimental import pallas as pl
from jax.experimental.pallas import tpu as pltpu

_NC = 6
_TOPK = 10
_ALPHA = 0.5
_BETA = 6.0
_BOX_W = 7.5
_CLS_W = 0.5
_INNER_W = 0.5
_SCALE = 0.7
_LEVELS = [(80, 80), (40, 40), (20, 20)]
_A = sum(h * w for h, w in _LEVELS)
_M = 40
_EPS = 1e-9


def _anchors_np():
    pts = []
    for (H, W) in _LEVELS:
        gy, gx = np.meshgrid(np.arange(H), np.arange(W), indexing="ij")
        ax = (gx + 0.5) / W
        ay = (gy + 0.5) / H
        pts.append(np.stack([ax.reshape(-1), ay.reshape(-1)], 0))
    return np.concatenate(pts, 1).astype(np.float32)


def _loss_kernel(cls_ref, box_ref, tgt_ref, anc_ref, out_ref):
    b = pl.program_id(0)
    nb = pl.num_programs(0)

    @pl.when(b == 0)
    def _init():
        out_ref[...] = jnp.zeros_like(out_ref)

    x = cls_ref[0]
    ps = jax.nn.sigmoid(x)
    bx = jnp.clip(jax.nn.sigmoid(box_ref[0]), 1e-4, 1.0 - 1e-4)
    pc = bx[0:2, :]
    ph2 = bx[2:4, :] / 2
    plo = pc - ph2
    phi = pc + ph2
    px1, py1 = plo[0:1, :], plo[1:2, :]
    px2, py2 = phi[0:1, :], phi[1:2, :]

    tgt = tgt_ref[0]
    gl = jnp.clip(jnp.floor(tgt[:, 0:1]), 0.0, _NC - 1.0)
    gcx, gcy = tgt[:, 1:2], tgt[:, 2:3]
    gw, gh = tgt[:, 3:4], tgt[:, 4:5]
    gx1, gy1 = gcx - gw / 2, gcy - gh / 2
    gx2, gy2 = gcx + gw / 2, gcy + gh / 2

    ax = anc_ref[0, 0:1, :]
    ay = anc_ref[0, 1:2, :]

    d1 = jnp.minimum(ax - gx1, ay - gy1)
    d2 = jnp.minimum(gx2 - ax, gy2 - ay)
    mask_in = jnp.minimum(d1, d2) > _EPS

    ltx = jnp.maximum(gx1, px1)
    lty = jnp.maximum(gy1, py1)
    rbx = jnp.minimum(gx2, px2)
    rby = jnp.minimum(gy2, py2)
    iw = jnp.clip(rbx - ltx, 0.0, None)
    ih = jnp.clip(rby - lty, 0.0, None)
    inter = iw * ih
    a1 = jnp.clip(gx2 - gx1, 0.0, None) * jnp.clip(gy2 - gy1, 0.0, None)
    a2 = jnp.clip(px2 - px1, 0.0, None) * jnp.clip(py2 - py1, 0.0, None)
    overlaps = inter / (a1 + a2 - inter + 1e-7)

    sq = jnp.sqrt(ps)
    sqscore = jnp.where(
        gl == 0.0, sq[0:1, :],
        jnp.where(gl == 1.0, sq[1:2, :],
                  jnp.where(gl == 2.0, sq[2:3, :],
                            jnp.where(gl == 3.0, sq[3:4, :],
                                      jnp.where(gl == 4.0, sq[4:5, :],
                                                sq[5:6, :])))))

    o2 = overlaps * overlaps
    align = sqscore * (o2 * o2 * o2)
    metric = jnp.where(mask_in, align, 0.0)

    c = jnp.max(metric, axis=1, keepdims=True)
    for _ in range(_TOPK - 1):
        c = jnp.max(jnp.where(metric >= c, -1.0, metric),
                    axis=1, keepdims=True)
    mask_pos0 = ((metric >= c) & (metric > _EPS)).astype(jnp.float32)

    fg_cnt = jnp.sum(mask_pos0, axis=0, keepdims=True)
    multi = fg_cnt > 1.0
    fg = fg_cnt > 0.0
    fgf = fg.astype(jnp.float32)

    mo = jnp.max(overlaps, axis=0, keepdims=True)
    is_max = (overlaps == mo).astype(jnp.float32)
    mask_pos = jnp.where(multi, is_max, mask_pos0)

    t_lab = jnp.sum(mask_pos * gl, axis=0, keepdims=True)
    tcx = jnp.sum(mask_pos * gcx, axis=0, keepdims=True)
    tcy = jnp.sum(mask_pos * gcy, axis=0, keepdims=True)
    tw = jnp.sum(mask_pos * gw, axis=0, keepdims=True)
    th = jnp.sum(mask_pos * gh, axis=0, keepdims=True)

    align_pos = align * mask_pos
    pos_align = jnp.max(align_pos, axis=1, keepdims=True)
    pos_ov = jnp.max(overlaps * mask_pos, axis=1, keepdims=True)
    ratio = pos_ov / (pos_align + _EPS)
    norm_max = jnp.max(align_pos * ratio, axis=0, keepdims=True)

    tmag = fgf * norm_max
    iota_c = jax.lax.broadcasted_iota(jnp.int32, (_NC, _A), 0).astype(jnp.float32)
    t_mat = jnp.where(iota_c == t_lab, tmag, 0.0)
    bce = jnp.clip(x, 0.0, None) - x * t_mat + jnp.log1p(jnp.exp(-jnp.abs(x)))
    bce_sum = jnp.sum(bce)

    def iou_2a(lo1, hi1, lo2, hi2):
        jwh = jnp.clip(jnp.minimum(hi1, hi2) - jnp.maximum(lo1, lo2), 0.0, None)
        ji = jwh[0:1, :] * jwh[1:2, :]
        wh1 = jnp.clip(hi1 - lo1, 0.0, None)
        wh2 = jnp.clip(hi2 - lo2, 0.0, None)
        aa = wh1[0:1, :] * wh1[1:2, :]
        ab = wh2[0:1, :] * wh2[1:2, :]
        return ji / (aa + ab - ji + 1e-7)

    tc = jnp.concatenate([tcx, tcy], 0)
    th2 = jnp.concatenate([tw, th], 0) / 2
    tlo, thi = tc - th2, tc + th2
    iou = iou_2a(plo, phi, tlo, thi)
    ph2s, th2s = ph2 * _SCALE, th2 * _SCALE
    inner = iou_2a(pc - ph2s, pc + ph2s, tc - th2s, tc + th2s)
    per = (1.0 - iou) + _INNER_W * (1.0 - inner)
    box_sum = jnp.sum(per * fgf)
    fg_sum = jnp.sum(fgf)

    li = jax.lax.broadcasted_iota(jnp.int32, (1, 128), 1)
    contrib = (jnp.where(li == 0, fg_sum, 0.0)
               + jnp.where(li == 1, bce_sum, 0.0)
               + jnp.where(li == 2, box_sum, 0.0))
    out_ref[...] += contrib

    @pl.when(b == nb - 1)
    def _fin():
        acc = out_ref[...]
        num_fg = jnp.clip(acc[0:1, 0:1], 1.0, None)
        cls_loss = acc[0:1, 1:2] / num_fg
        box_loss = acc[0:1, 2:3] / num_fg
        total = _CLS_W * cls_loss + _BOX_W * box_loss
        fin = (jnp.where(li == 3, total, 0.0)
               + jnp.where(li == 4, cls_loss, 0.0)
               + jnp.where(li == 5, box_loss, 0.0))
        out_ref[...] = acc + fin


@jax.jit
def kernel(p3, p4, p5, targets):
    B = p3.shape[0]
    cls_in = jnp.concatenate(
        [p[:, :_NC].reshape(B, _NC, -1) for p in (p3, p4, p5)], axis=2)
    box_in = jnp.concatenate(
        [p[:, _NC + 1:].reshape(B, 4, -1) for p in (p3, p4, p5)], axis=2)
    anc = jnp.asarray(_anchors_np())[None]

    out = pl.pallas_call(
        _loss_kernel,
        grid=(B,),
        in_specs=[
            pl.BlockSpec((1, _NC, _A), lambda b: (b, 0, 0)),
            pl.BlockSpec((1, 4, _A), lambda b: (b, 0, 0)),
            pl.BlockSpec((1, _M, 5), lambda b: (b, 0, 0)),
            pl.BlockSpec((1, 2, _A), lambda b: (0, 0, 0)),
        ],
        out_specs=pl.BlockSpec((1, 128), lambda b: (0, 0)),
        out_shape=jax.ShapeDtypeStruct((1, 128), jnp.float32),
    )(cls_in, box_in, targets, anc)

    total = out[0, 3]
    cls_loss = out[0, 4]
    box_loss = out[0, 5]
    return total, cls_loss, box_loss


# speedup vs baseline: 1.1160x; 1.0214x over previous
import functools

import jax
import jax.numpy as jnp
import numpy as np
from jax.experimental import pallas as pl
from jax.experimental.pallas import tpu as pltpu

_NC = 6
_TOPK = 10
_ALPHA = 0.5
_BETA = 6.0
_BOX_W = 7.5
_CLS_W = 0.5
_INNER_W = 0.5
_SCALE = 0.7
_LEVELS = [(80, 80), (40, 40), (20, 20)]
_A = sum(h * w for h, w in _LEVELS)
_M = 40
_EPS = 1e-9


def _anchors_np():
    pts = []
    for (H, W) in _LEVELS:
        gy, gx = np.meshgrid(np.arange(H), np.arange(W), indexing="ij")
        ax = (gx + 0.5) / W
        ay = (gy + 0.5) / H
        pts.append(np.stack([ax.reshape(-1), ay.reshape(-1)], 0))
    return np.concatenate(pts, 1).astype(np.float32)


def _loss_kernel(cls_ref, box_ref, tgt_ref, anc_ref, out_ref):
    b = pl.program_id(0)
    nb = pl.num_programs(0)

    @pl.when(b == 0)
    def _init():
        out_ref[...] = jnp.zeros_like(out_ref)

    x = cls_ref[0]
    ps = jax.nn.sigmoid(x)
    bx = jnp.clip(jax.nn.sigmoid(box_ref[0]), 1e-4, 1.0 - 1e-4)
    pc = bx[0:2, :]
    ph2 = bx[2:4, :] / 2
    plo = pc - ph2
    phi = pc + ph2
    px1, py1 = plo[0:1, :], plo[1:2, :]
    px2, py2 = phi[0:1, :], phi[1:2, :]

    tgt = tgt_ref[0]
    gl = jnp.clip(jnp.floor(tgt[:, 0:1]), 0.0, _NC - 1.0)
    gcx, gcy = tgt[:, 1:2], tgt[:, 2:3]
    gw, gh = tgt[:, 3:4], tgt[:, 4:5]
    gx1, gy1 = gcx - gw / 2, gcy - gh / 2
    gx2, gy2 = gcx + gw / 2, gcy + gh / 2

    ax = anc_ref[0, 0:1, :]
    ay = anc_ref[0, 1:2, :]

    d1 = jnp.minimum(ax - gx1, ay - gy1)
    d2 = jnp.minimum(gx2 - ax, gy2 - ay)
    mask_in = jnp.minimum(d1, d2) > _EPS

    ltx = jnp.maximum(gx1, px1)
    lty = jnp.maximum(gy1, py1)
    rbx = jnp.minimum(gx2, px2)
    rby = jnp.minimum(gy2, py2)
    iw = jnp.clip(rbx - ltx, 0.0, None)
    ih = jnp.clip(rby - lty, 0.0, None)
    inter = iw * ih
    a1 = jnp.clip(gx2 - gx1, 0.0, None) * jnp.clip(gy2 - gy1, 0.0, None)
    a2 = jnp.clip(px2 - px1, 0.0, None) * jnp.clip(py2 - py1, 0.0, None)
    overlaps = inter / (a1 + a2 - inter + 1e-7)

    sq = jnp.sqrt(ps)
    sqscore = jnp.where(
        gl == 0.0, sq[0:1, :],
        jnp.where(gl == 1.0, sq[1:2, :],
                  jnp.where(gl == 2.0, sq[2:3, :],
                            jnp.where(gl == 3.0, sq[3:4, :],
                                      jnp.where(gl == 4.0, sq[4:5, :],
                                                sq[5:6, :])))))

    o2 = overlaps * overlaps
    align = sqscore * (o2 * o2 * o2)
    metric = jnp.where(mask_in, align, 0.0)

    c = jnp.max(metric, axis=1, keepdims=True)
    for _ in range(_TOPK - 1):
        c = jnp.max(jnp.where(metric >= c, -1.0, metric),
                    axis=1, keepdims=True)
    mask_pos0 = ((metric >= c) & (metric > _EPS)).astype(jnp.float32)

    fg_cnt = jnp.sum(mask_pos0, axis=0, keepdims=True)
    multi = fg_cnt > 1.0
    fg = fg_cnt > 0.0
    fgf = fg.astype(jnp.float32)

    mo = jnp.max(overlaps, axis=0, keepdims=True)
    is_max = (overlaps == mo).astype(jnp.float32)
    mask_pos = jnp.where(multi, is_max, mask_pos0)

    t_lab = jnp.sum(mask_pos * gl, axis=0, keepdims=True)
    tcx = jnp.sum(mask_pos * gcx, axis=0, keepdims=True)
    tcy = jnp.sum(mask_pos * gcy, axis=0, keepdims=True)
    tw = jnp.sum(mask_pos * gw, axis=0, keepdims=True)
    th = jnp.sum(mask_pos * gh, axis=0, keepdims=True)

    align_pos = align * mask_pos
    ovmask = overlaps * mask_pos
    pos_align = jnp.max(align_pos, axis=1, keepdims=True)
    pos_ov = jnp.max(ovmask, axis=1, keepdims=True)
    ratio = pos_ov / (pos_align + _EPS)
    norm_max = jnp.max(align_pos * ratio, axis=0, keepdims=True)

    iota_c = jax.lax.broadcasted_iota(jnp.int32, (_NC, _A), 0).astype(jnp.float32)
    t_mat = jnp.where(iota_c == t_lab, norm_max, 0.0)
    bce = jnp.clip(x, 0.0, None) - x * t_mat + jnp.log1p(jnp.exp(-jnp.abs(x)))
    bce_sum = jnp.sum(bce)

    tc = jnp.concatenate([tcx, tcy], 0)
    th2 = jnp.concatenate([tw, th], 0) / 2
    ph2s, th2s = ph2 * _SCALE, th2 * _SCALE
    plo_s, phi_s = pc - ph2s, pc + ph2s
    tlo_s, thi_s = tc - th2s, tc + th2s
    jwh = jnp.clip(jnp.minimum(phi_s, thi_s) - jnp.maximum(plo_s, tlo_s),
                   0.0, None)
    ji = jwh[0:1, :] * jwh[1:2, :]
    wh1 = jnp.clip(phi_s - plo_s, 0.0, None)
    wh2 = jnp.clip(thi_s - tlo_s, 0.0, None)
    aa = wh1[0:1, :] * wh1[1:2, :]
    ab = wh2[0:1, :] * wh2[1:2, :]
    inner = ji / (aa + ab - ji + 1e-7)

    fg_sum = jnp.sum(fgf)
    box_sum = ((1.0 + _INNER_W) * fg_sum - jnp.sum(ovmask)
               - _INNER_W * jnp.sum(inner))

    li = jax.lax.broadcasted_iota(jnp.int32, (1, 128), 1)
    contrib = (jnp.where(li == 0, fg_sum, 0.0)
               + jnp.where(li == 1, bce_sum, 0.0)
               + jnp.where(li == 2, box_sum, 0.0))
    out_ref[...] += contrib

    @pl.when(b == nb - 1)
    def _fin():
        acc = out_ref[...]
        num_fg = jnp.clip(acc[0:1, 0:1], 1.0, None)
        cls_loss = acc[0:1, 1:2] / num_fg
        box_loss = acc[0:1, 2:3] / num_fg
        total = _CLS_W * cls_loss + _BOX_W * box_loss
        fin = (jnp.where(li == 3, total, 0.0)
               + jnp.where(li == 4, cls_loss, 0.0)
               + jnp.where(li == 5, box_loss, 0.0))
        out_ref[...] = acc + fin


@jax.jit
def kernel(p3, p4, p5, targets):
    B = p3.shape[0]
    cls_in = jnp.concatenate(
        [p[:, :_NC].reshape(B, _NC, -1) for p in (p3, p4, p5)], axis=2)
    box_in = jnp.concatenate(
        [p[:, _NC + 1:].reshape(B, 4, -1) for p in (p3, p4, p5)], axis=2)
    anc = jnp.asarray(_anchors_np())[None]

    out = pl.pallas_call(
        _loss_kernel,
        grid=(B,),
        in_specs=[
            pl.BlockSpec((1, _NC, _A), lambda b: (b, 0, 0)),
            pl.BlockSpec((1, 4, _A), lambda b: (b, 0, 0)),
            pl.BlockSpec((1, _M, 5), lambda b: (b, 0, 0)),
            pl.BlockSpec((1, 2, _A), lambda b: (0, 0, 0)),
        ],
        out_specs=pl.BlockSpec((1, 128), lambda b: (0, 0)),
        out_shape=jax.ShapeDtypeStruct((1, 128), jnp.float32),
    )(cls_in, box_in, targets, anc)

    total = out[0, 3]
    cls_loss = out[0, 4]
    box_loss = out[0, 5]
    return total, cls_loss, box_loss


# speedup vs baseline: 1.1452x; 1.0262x over previous
import functools

import jax
import jax.numpy as jnp
import numpy as np
from jax.experimental import pallas as pl
from jax.experimental.pallas import tpu as pltpu

_NC = 6
_TOPK = 10
_ALPHA = 0.5
_BETA = 6.0
_BOX_W = 7.5
_CLS_W = 0.5
_INNER_W = 0.5
_SCALE = 0.7
_LEVELS = [(80, 80), (40, 40), (20, 20)]
_A = sum(h * w for h, w in _LEVELS)
_M = 40
_EPS = 1e-9


def _anchors_np():
    pts = []
    for (H, W) in _LEVELS:
        gy, gx = np.meshgrid(np.arange(H), np.arange(W), indexing="ij")
        ax = (gx + 0.5) / W
        ay = (gy + 0.5) / H
        pts.append(np.stack([ax.reshape(-1), ay.reshape(-1)], 0))
    return np.concatenate(pts, 1).astype(np.float32)


def _loss_kernel(cls_ref, box_ref, tgt_ref, anc_ref, out_ref):
    b = pl.program_id(0)
    nb = pl.num_programs(0)

    @pl.when(b == 0)
    def _init():
        out_ref[...] = jnp.zeros_like(out_ref)

    x = cls_ref[0]
    ps = jax.nn.sigmoid(x)
    bx = jnp.clip(jax.nn.sigmoid(box_ref[0]), 1e-4, 1.0 - 1e-4)
    pc = bx[0:2, :]
    ph2 = bx[2:4, :] / 2
    plo = pc - ph2
    phi = pc + ph2
    px1, py1 = plo[0:1, :], plo[1:2, :]
    px2, py2 = phi[0:1, :], phi[1:2, :]

    tgt = tgt_ref[0]
    gl = jnp.clip(jnp.floor(tgt[:, 0:1]), 0.0, _NC - 1.0)
    gcx, gcy = tgt[:, 1:2], tgt[:, 2:3]
    gw, gh = tgt[:, 3:4], tgt[:, 4:5]
    gx1, gy1 = gcx - gw / 2, gcy - gh / 2
    gx2, gy2 = gcx + gw / 2, gcy + gh / 2

    ax = anc_ref[0, 0:1, :]
    ay = anc_ref[0, 1:2, :]

    mask_in = ((ax > gx1 + _EPS) & (ay > gy1 + _EPS)
               & (ax < gx2 - _EPS) & (ay < gy2 - _EPS))

    ltx = jnp.maximum(gx1, px1)
    lty = jnp.maximum(gy1, py1)
    rbx = jnp.minimum(gx2, px2)
    rby = jnp.minimum(gy2, py2)
    iw = jnp.clip(rbx - ltx, 0.0, None)
    ih = jnp.clip(rby - lty, 0.0, None)
    inter = iw * ih
    a1 = jnp.clip(gx2 - gx1, 0.0, None) * jnp.clip(gy2 - gy1, 0.0, None)
    a2 = jnp.clip(px2 - px1, 0.0, None) * jnp.clip(py2 - py1, 0.0, None)
    overlaps = inter / (a1 + a2 - inter + 1e-7)

    sq = jnp.sqrt(ps)
    sqscore = jnp.where(
        gl == 0.0, sq[0:1, :],
        jnp.where(gl == 1.0, sq[1:2, :],
                  jnp.where(gl == 2.0, sq[2:3, :],
                            jnp.where(gl == 3.0, sq[3:4, :],
                                      jnp.where(gl == 4.0, sq[4:5, :],
                                                sq[5:6, :])))))

    o2 = overlaps * overlaps
    align = sqscore * (o2 * o2 * o2)
    metric = jnp.where(mask_in, align, 0.0)

    c = jnp.max(metric, axis=1, keepdims=True)
    for _ in range(_TOPK - 1):
        c = jnp.max(jnp.where(metric >= c, -1.0, metric),
                    axis=1, keepdims=True)
    mask_pos0 = ((metric >= c) & (metric > _EPS)).astype(jnp.float32)

    fg_cnt = jnp.sum(mask_pos0, axis=0, keepdims=True)
    multi = fg_cnt > 1.0
    fg = fg_cnt > 0.0
    fgf = fg.astype(jnp.float32)

    mo = jnp.max(overlaps, axis=0, keepdims=True)
    is_max = (overlaps == mo).astype(jnp.float32)
    mask_pos = jnp.where(multi, is_max, mask_pos0)

    t_lab = jnp.sum(mask_pos * gl, axis=0, keepdims=True)
    tcx = jnp.sum(mask_pos * gcx, axis=0, keepdims=True)
    tcy = jnp.sum(mask_pos * gcy, axis=0, keepdims=True)
    tw = jnp.sum(mask_pos * gw, axis=0, keepdims=True)
    th = jnp.sum(mask_pos * gh, axis=0, keepdims=True)

    align_pos = align * mask_pos
    ovmask = overlaps * mask_pos
    pos_align = jnp.max(align_pos, axis=1, keepdims=True)
    pos_ov = jnp.max(ovmask, axis=1, keepdims=True)
    ratio = pos_ov / (pos_align + _EPS)
    norm_max = jnp.max(align_pos * ratio, axis=0, keepdims=True)

    iota_c = jax.lax.broadcasted_iota(jnp.int32, (_NC, _A), 0).astype(jnp.float32)
    t_mat = jnp.where(iota_c == t_lab, norm_max, 0.0)
    bce = jnp.clip(x, 0.0, None) - x * t_mat + jnp.log1p(jnp.exp(-jnp.abs(x)))
    bce_sum = jnp.sum(bce)

    tc = jnp.concatenate([tcx, tcy], 0)
    th2 = jnp.concatenate([tw, th], 0) / 2
    ph2s, th2s = ph2 * _SCALE, th2 * _SCALE
    plo_s, phi_s = pc - ph2s, pc + ph2s
    tlo_s, thi_s = tc - th2s, tc + th2s
    jwh = jnp.clip(jnp.minimum(phi_s, thi_s) - jnp.maximum(plo_s, tlo_s),
                   0.0, None)
    ji = jwh[0:1, :] * jwh[1:2, :]
    wh1 = jnp.clip(phi_s - plo_s, 0.0, None)
    wh2 = jnp.clip(thi_s - tlo_s, 0.0, None)
    aa = wh1[0:1, :] * wh1[1:2, :]
    ab = wh2[0:1, :] * wh2[1:2, :]
    inner = ji / (aa + ab - ji + 1e-7)

    fg_sum = jnp.sum(fgf)
    box_sum = ((1.0 + _INNER_W) * fg_sum - jnp.sum(ovmask)
               - _INNER_W * jnp.sum(inner))

    li = jax.lax.broadcasted_iota(jnp.int32, (1, 128), 1)
    contrib = (jnp.where(li == 0, fg_sum, 0.0)
               + jnp.where(li == 1, bce_sum, 0.0)
               + jnp.where(li == 2, box_sum, 0.0))
    out_ref[...] += contrib

    @pl.when(b == nb - 1)
    def _fin():
        acc = out_ref[...]
        num_fg = jnp.clip(acc[0:1, 0:1], 1.0, None)
        cls_loss = acc[0:1, 1:2] / num_fg
        box_loss = acc[0:1, 2:3] / num_fg
        total = _CLS_W * cls_loss + _BOX_W * box_loss
        fin = (jnp.where(li == 3, total, 0.0)
               + jnp.where(li == 4, cls_loss, 0.0)
               + jnp.where(li == 5, box_loss, 0.0))
        out_ref[...] = acc + fin


@jax.jit
def kernel(p3, p4, p5, targets):
    B = p3.shape[0]
    cls_in = jnp.concatenate(
        [p[:, :_NC].reshape(B, _NC, -1) for p in (p3, p4, p5)], axis=2)
    box_in = jnp.concatenate(
        [p[:, _NC + 1:].reshape(B, 4, -1) for p in (p3, p4, p5)], axis=2)
    anc = jnp.asarray(_anchors_np())[None]

    out = pl.pallas_call(
        _loss_kernel,
        grid=(B,),
        in_specs=[
            pl.BlockSpec((1, _NC, _A), lambda b: (b, 0, 0)),
            pl.BlockSpec((1, 4, _A), lambda b: (b, 0, 0)),
            pl.BlockSpec((1, _M, 5), lambda b: (b, 0, 0)),
            pl.BlockSpec((1, 2, _A), lambda b: (0, 0, 0)),
        ],
        out_specs=pl.BlockSpec((1, 128), lambda b: (0, 0)),
        out_shape=jax.ShapeDtypeStruct((1, 128), jnp.float32),
    )(cls_in, box_in, targets, anc)

    total = out[0, 3]
    cls_loss = out[0, 4]
    box_loss = out[0, 5]
    return total, cls_loss, box_loss
